# Initial kernel scaffold; baseline (speedup 1.0000x reference)
import jax
import jax.numpy as jnp
from jax.experimental import pallas as pl


def kernel(x, edge_index, edge_type, relation_weights, relation_bias, self_loop_w, self_loop_b, ln_weight, ln_bias):
    raise NotImplementedError("write your pallas kernel here")


# speedup vs baseline: 23.4563x; 23.4563x over previous
import functools

import jax
import jax.numpy as jnp
from jax import lax
from jax.experimental import pallas as pl
from jax.experimental.pallas import tpu as pltpu
from jax.experimental.pallas import tpu_sc as plsc

IN_DIM = 128
OUT_DIM = 128
NUM_RELATIONS = 10
N_NODES = 10000
N_EDGES = 320000

NC = 2
NS = 16
NW = NC * NS
CHUNK = 128
CHUNKS_PER_W = (N_EDGES + NW * CHUNK - 1) // (NW * CHUNK)
EPW = CHUNKS_PER_W * CHUNK
E_PAD = EPW * NW
ACC_ROWS = 10240
ROWS_PER_TILE = ACC_ROWS // NS
PAD_DST = N_NODES

_ROW_BLK = 1000


def _relation_transform(x, rw, rb):
    def body(x_ref, w_ref, b_ref, o_ref):
        o_ref[...] = (
            jnp.dot(x_ref[...], w_ref[0], preferred_element_type=jnp.float32)
            + b_ref[0]
        )

    nblk = N_NODES // _ROW_BLK
    rb = rb.reshape(NUM_RELATIONS, 1, OUT_DIM)
    return pl.pallas_call(
        body,
        grid=(NUM_RELATIONS, nblk),
        in_specs=[
            pl.BlockSpec((_ROW_BLK, IN_DIM), lambda r, i: (i, 0)),
            pl.BlockSpec((1, IN_DIM, OUT_DIM), lambda r, i: (r, 0, 0)),
            pl.BlockSpec((1, 1, OUT_DIM), lambda r, i: (r, 0, 0)),
        ],
        out_specs=pl.BlockSpec((_ROW_BLK, OUT_DIM), lambda r, i: (r * nblk + i, 0)),
        out_shape=jax.ShapeDtypeStruct((NUM_RELATIONS * N_NODES, OUT_DIM), jnp.float32),
    )(x, rw, rb)


def _make_sc_scatter():
    mesh = plsc.VectorSubcoreMesh(core_axis_name="c", subcore_axis_name="s")

    @functools.partial(
        pl.kernel,
        out_type=jax.ShapeDtypeStruct((NC, ACC_ROWS, OUT_DIM), jnp.float32),
        mesh=mesh,
        scratch_types=[
            pltpu.VMEM((CHUNK,), jnp.int32),
            pltpu.VMEM((CHUNK,), jnp.int32),
            pltpu.VMEM((CHUNK,), jnp.int32),
            pltpu.VMEM((1, CHUNK), jnp.int32),
            pltpu.VMEM((CHUNK, OUT_DIM), jnp.float32),
            pltpu.VMEM_SHARED((ACC_ROWS, OUT_DIM), jnp.float32),
            pltpu.SemaphoreType.DMA,
        ],
    )
    def sc_scatter(src_hbm, typ_hbm, dst_hbm, table_hbm, out_hbm,
                   srcb, typb, idxb, dstb, rows, acc, sem):
        c = lax.axis_index("c")
        s = lax.axis_index("s")
        wid = c * NS + s
        base = wid * EPW

        def zero_row(i, carry):
            for j in range(OUT_DIM // 16):
                rows[i, pl.ds(j * 16, 16)] = jnp.zeros((16,), jnp.float32)
            return carry

        lax.fori_loop(0, CHUNK, zero_row, 0)
        for k in range(ROWS_PER_TILE // CHUNK):
            pltpu.sync_copy(rows, acc.at[pl.ds(s * ROWS_PER_TILE + k * CHUNK, CHUNK)])
        plsc.subcore_barrier()

        def chunk_body(t, carry):
            off = base + t * CHUNK
            pltpu.sync_copy(src_hbm.at[pl.ds(off, CHUNK)], srcb)
            pltpu.sync_copy(typ_hbm.at[pl.ds(off, CHUNK)], typb)
            pltpu.sync_copy(dst_hbm.at[pl.ds(off, CHUNK)], dstb.at[0])
            for j in range(CHUNK // 16):
                sl = pl.ds(j * 16, 16)
                idxb[sl] = typb[sl] * N_NODES + srcb[sl]
            pltpu.async_copy(table_hbm.at[idxb], rows, sem).wait()
            pltpu.sync_copy(rows, acc.at[dstb.at[0]], add=True)
            return carry

        lax.fori_loop(0, CHUNKS_PER_W, chunk_body, 0)
        plsc.subcore_barrier()

        pltpu.sync_copy(
            acc.at[pl.ds(s * ROWS_PER_TILE, ROWS_PER_TILE)],
            out_hbm.at[c, pl.ds(s * ROWS_PER_TILE, ROWS_PER_TILE)],
        )

    return sc_scatter


_sc_scatter = _make_sc_scatter()


def _epilogue(x, wt, b, partial, gamma, beta):
    def body(x_ref, wt_ref, b_ref, p0_ref, p1_ref, g_ref, be_ref, o_ref):
        h = jnp.dot(x_ref[...], wt_ref[...], preferred_element_type=jnp.float32)
        h = h + b_ref[0][None, :] + p0_ref[0] + p1_ref[0]
        mu = jnp.mean(h, axis=1, keepdims=True)
        d = h - mu
        var = jnp.mean(d * d, axis=1, keepdims=True)
        o_ref[...] = d * lax.rsqrt(var + 1e-5) * g_ref[0][None, :] + be_ref[0][None, :]

    nblk = N_NODES // _ROW_BLK
    return pl.pallas_call(
        body,
        grid=(nblk,),
        in_specs=[
            pl.BlockSpec((_ROW_BLK, IN_DIM), lambda i: (i, 0)),
            pl.BlockSpec((IN_DIM, OUT_DIM), lambda i: (0, 0)),
            pl.BlockSpec((1, OUT_DIM), lambda i: (0, 0)),
            pl.BlockSpec((1, _ROW_BLK, OUT_DIM), lambda i: (0, i, 0)),
            pl.BlockSpec((1, _ROW_BLK, OUT_DIM), lambda i: (1, i, 0)),
            pl.BlockSpec((1, OUT_DIM), lambda i: (0, 0)),
            pl.BlockSpec((1, OUT_DIM), lambda i: (0, 0)),
        ],
        out_specs=pl.BlockSpec((_ROW_BLK, OUT_DIM), lambda i: (i, 0)),
        out_shape=jax.ShapeDtypeStruct((N_NODES, OUT_DIM), jnp.float32),
    )(x, wt, b, partial, partial, gamma, beta)


def kernel(x, edge_index, edge_type, relation_weights, relation_bias,
           self_loop_w, self_loop_b, ln_weight, ln_bias):
    src = edge_index[0].astype(jnp.int32)
    dst = edge_index[1].astype(jnp.int32)
    typ = edge_type.astype(jnp.int32)

    npad = E_PAD - N_EDGES
    src_p = jnp.concatenate([src, jnp.zeros((npad,), jnp.int32)])
    typ_p = jnp.concatenate([typ, jnp.zeros((npad,), jnp.int32)])
    dst_p = jnp.concatenate([dst, jnp.full((npad,), PAD_DST, jnp.int32)])

    table = _relation_transform(x, relation_weights, relation_bias)
    partial = _sc_scatter(src_p, typ_p, dst_p, table)

    return _epilogue(
        x,
        self_loop_w.T,
        self_loop_b.reshape(1, OUT_DIM),
        partial,
        ln_weight.reshape(1, OUT_DIM),
        ln_bias.reshape(1, OUT_DIM),
    )
